# baseline (speedup 1.0000x reference)
import functools

import jax
import jax.numpy as jnp
from jax import lax
from jax.experimental import pallas as pl
from jax.experimental.pallas import tpu as pltpu
from jax.experimental.pallas import tpu_sc as plsc

B = 16384
F = 26
V = 100000
D = 16

NC = 2
NS = 16
NW = NC * NS
ROWS_W = B // NW
CHUNK_R = 64
N_CHUNK = ROWS_W // CHUNK_R
G_CHUNK = CHUNK_R * F
SUB = 128
N_SUB = G_CHUNK // SUB


def _sc_bi_kernel(idx_hbm, tab_hbm, bi_hbm, psum_hbm, psq_hbm,
                  idx_v, rows0, rows1, bi_v, part_v, sem0, sem1):
    wid = lax.axis_index("s") * NC + lax.axis_index("c")
    base = wid * ROWS_W

    pltpu.sync_copy(idx_hbm.at[pl.ds(base * F, ROWS_W * F)], idx_v)

    rows = (rows0, rows1)
    sems = (sem0, sem1)

    def fire(c, slot):
        descs = []
        for s in range(N_SUB):
            off = c * G_CHUNK + s * SUB
            descs.append(pltpu.async_copy(
                tab_hbm.at[idx_v.at[pl.ds(off, SUB)]],
                rows[slot].at[pl.ds(s * SUB, SUB)],
                sems[slot]))
        return descs

    zeros = jnp.zeros((D,), jnp.float32)
    psum = zeros
    psq = zeros

    inflight = {0: fire(0, 0), 1: None}
    for c in range(N_CHUNK):
        slot = c % 2
        if c + 1 < N_CHUNK:
            inflight[1 - slot] = fire(c + 1, 1 - slot)
        for d_ in inflight[slot]:
            d_.wait()
        buf = rows[slot]

        @pl.loop(0, CHUNK_R, init_carry=(psum, psq))
        def _row(r, carry):
            ps, pq = carry
            e = buf[r * F]
            s = e
            sq = e * e
            for f in range(1, F):
                e = buf[r * F + f]
                s = s + e
                sq = sq + e * e
            bi = 0.5 * (s * s - sq)
            bi_v[c * CHUNK_R + r] = bi
            return ps + bi, pq + bi * bi

        psum, psq = _row

    part_v[0] = psum
    part_v[1] = psq
    pltpu.sync_copy(bi_v, bi_hbm.at[pl.ds(base, ROWS_W)])
    pltpu.sync_copy(part_v.at[0], psum_hbm.at[wid])
    pltpu.sync_copy(part_v.at[1], psq_hbm.at[wid])


_sc_bi = functools.partial(
    pl.kernel,
    out_type=[
        jax.ShapeDtypeStruct((B, D), jnp.float32),
        jax.ShapeDtypeStruct((NW, D), jnp.float32),
        jax.ShapeDtypeStruct((NW, D), jnp.float32),
    ],
    mesh=plsc.VectorSubcoreMesh(core_axis_name="c", subcore_axis_name="s"),
    scratch_types=[
        pltpu.VMEM((ROWS_W * F,), jnp.int32),
        pltpu.VMEM((G_CHUNK, D), jnp.float32),
        pltpu.VMEM((G_CHUNK, D), jnp.float32),
        pltpu.VMEM((ROWS_W, D), jnp.float32),
        pltpu.VMEM((2, D), jnp.float32),
        pltpu.SemaphoreType.DMA,
        pltpu.SemaphoreType.DMA,
    ],
    compiler_params=pltpu.CompilerParams(use_tc_tiling_on_sc=False),
)(_sc_bi_kernel)


def _tc_mlp_kernel(bi_ref, psum_ref, psq_ref, gamma_ref, beta_ref,
                   w1_ref, b1_ref, w2_ref, b2_ref, w3_ref, b3_ref, out_ref):
    inv_b = 1.0 / B
    mean = jnp.sum(psum_ref[...], axis=0, keepdims=True) * inv_b
    ex2 = jnp.sum(psq_ref[...], axis=0, keepdims=True) * inv_b
    var = ex2 - mean * mean
    scale = gamma_ref[...] * jax.lax.rsqrt(var + 1e-3)
    shift = beta_ref[...] - mean * scale
    x = bi_ref[...] * scale + shift
    h = jnp.dot(x, w1_ref[...], preferred_element_type=jnp.float32) + b1_ref[...]
    h = jnp.maximum(h, 0.0)
    h = jnp.dot(h, w2_ref[...], preferred_element_type=jnp.float32) + b2_ref[...]
    h = jnp.maximum(h, 0.0)
    o = jnp.dot(h, w3_ref[...], preferred_element_type=jnp.float32) + b3_ref[...]
    out_ref[...] = 1.0 / (1.0 + jnp.exp(-o))


def kernel(tables, gamma, beta, W1, b1, W2, b2, W3, b3, indices):
    tab_flat = tables.reshape(F * V, D)
    flat_idx = (indices.astype(jnp.int32)
                + (jnp.arange(F, dtype=jnp.int32) * V)[None, :]).reshape(B * F)

    bi, psum, psq = _sc_bi(flat_idx, tab_flat)

    out = pl.pallas_call(
        _tc_mlp_kernel,
        out_shape=jax.ShapeDtypeStruct((B, 1), jnp.float32),
    )(bi, psum, psq,
      gamma.reshape(1, D), beta.reshape(1, D),
      W1, b1.reshape(1, 256), W2, b2.reshape(1, 128), W3, b3.reshape(1, 1))
    return out
